# baseline (speedup 1.0000x reference)
import jax
import jax.numpy as jnp
from jax import lax
from jax.experimental import pallas as pl
from jax.experimental.pallas import tpu as pltpu
from jax.experimental.pallas import tpu_sc as plsc


def _lookup_body(idx_hbm, table_hbm, out_hbm, idx_s):
    pltpu.sync_copy(idx_hbm, idx_s)
    i = idx_s[0]
    pltpu.sync_copy(table_hbm.at[i], out_hbm)


def kernel(letter_idx, letter_embed):
    idx = jnp.asarray(letter_idx, jnp.int32).reshape(1)
    mesh = plsc.ScalarSubcoreMesh(axis_name="c", num_cores=1)
    lookup = pl.kernel(
        _lookup_body,
        out_type=jax.ShapeDtypeStruct((8,), jnp.float32),
        mesh=mesh,
        scratch_types=[
            pltpu.SMEM((1,), jnp.int32),
        ],
        compiler_params=pltpu.CompilerParams(
            use_tc_tiling_on_sc=False,
            skip_device_barrier=True,
        ),
    )
    return lookup(idx, letter_embed)
